# baseline (speedup 1.0000x reference)
import jax
import jax.numpy as jnp
from jax.experimental import pallas as pl
from jax.experimental.pallas import tpu as pltpu

_TILE = 512
_NCHUNK = 256


def _mlp_kernel(x_ref, w1_ref, b1_ref, w2_ref, b2_ref, w3_ref, b3_ref,
                o_ref, w1bf_ref):
    @pl.when(pl.program_id(0) == 0)
    def _():
        w1bf_ref[...] = w1_ref[...].T.astype(jnp.bfloat16)

    w2b = w2_ref[...].T.astype(jnp.bfloat16)
    w3b = w3_ref[...].T.astype(jnp.bfloat16)
    b1v = b1_ref[...].reshape(1, -1)
    b2v = b2_ref[...].reshape(1, -1)
    b3v = b3_ref[...].reshape(1, -1)

    nf = x_ref.shape[0]
    hid1 = w1bf_ref.shape[1]
    xr = jnp.concatenate(
        [x_ref[f].astype(jnp.bfloat16) for f in range(nf)], axis=1)
    h2 = None
    for start in range(0, hid1, _NCHUNK):
        width = min(_NCHUNK, hid1 - start)
        h1c = jnp.dot(xr, w1bf_ref[:, start:start + width],
                      preferred_element_type=jnp.float32)
        h1c = jnp.maximum(h1c + b1v[:, start:start + width],
                          0.0).astype(jnp.bfloat16)
        p = jnp.dot(h1c, w2b[start:start + width, :],
                    preferred_element_type=jnp.float32)
        h2 = p if h2 is None else h2 + p
    h2 = jnp.maximum(h2 + b2v, 0.0).astype(jnp.bfloat16)
    out = jnp.dot(h2, w3b, preferred_element_type=jnp.float32)
    o_ref[...] = out + b3v


def kernel(field, W1, b1, W2, b2, W3, b3):
    B = field.shape[0]
    nf, nl = field.shape[1], field.shape[2]
    in_dim = nf * nl
    hid1 = W1.shape[0]
    hid2 = W2.shape[0]

    ft = jnp.transpose(field, (1, 0, 2))

    grid = (B // _TILE,)
    out = pl.pallas_call(
        _mlp_kernel,
        grid=grid,
        in_specs=[
            pl.BlockSpec((nf, _TILE, nl), lambda i: (0, i, 0)),
            pl.BlockSpec((hid1, in_dim), lambda i: (0, 0)),
            pl.BlockSpec((hid1,), lambda i: (0,)),
            pl.BlockSpec((hid2, hid1), lambda i: (0, 0)),
            pl.BlockSpec((hid2,), lambda i: (0,)),
            pl.BlockSpec((1, hid2), lambda i: (0, 0)),
            pl.BlockSpec((1,), lambda i: (0,)),
        ],
        out_specs=pl.BlockSpec((_TILE, 1), lambda i: (i, 0)),
        out_shape=jax.ShapeDtypeStruct((B, 1), jnp.float32),
        scratch_shapes=[
            pltpu.VMEM((in_dim, hid1), jnp.bfloat16),
        ],
    )(ft, W1, b1, W2, b2, W3, b3)
    return out


# speedup vs baseline: 1.0504x; 1.0504x over previous
import jax
import jax.numpy as jnp
from jax.experimental import pallas as pl
from jax.experimental.pallas import tpu as pltpu

_TILE = 512


def _mlp_kernel(x_ref, w1_ref, b1_ref, w2_ref, b2_ref, w3_ref, b3_ref,
                o_ref, w1bf_ref):
    @pl.when(pl.program_id(0) == 0)
    def _():
        w1bf_ref[...] = w1_ref[...].T.astype(jnp.bfloat16)

    w2b = w2_ref[...].T.astype(jnp.bfloat16)
    w3b = w3_ref[...].T.astype(jnp.bfloat16)
    b1v = b1_ref[...].reshape(1, -1)
    b2v = b2_ref[...].reshape(1, -1)
    b3v = b3_ref[...].reshape(1, -1)

    nf = x_ref.shape[0]
    xr = jnp.concatenate(
        [x_ref[f].astype(jnp.bfloat16) for f in range(nf)], axis=1)
    h1 = jnp.dot(xr, w1bf_ref[...], preferred_element_type=jnp.float32)
    h1 = jnp.maximum(h1 + b1v, 0.0).astype(jnp.bfloat16)
    h2 = jnp.dot(h1, w2b, preferred_element_type=jnp.float32)
    h2 = jnp.maximum(h2 + b2v, 0.0).astype(jnp.bfloat16)
    out = jnp.dot(h2, w3b, preferred_element_type=jnp.float32)
    o_ref[...] = out + b3v


def kernel(field, W1, b1, W2, b2, W3, b3):
    B = field.shape[0]
    nf, nl = field.shape[1], field.shape[2]
    in_dim = nf * nl
    hid1 = W1.shape[0]
    hid2 = W2.shape[0]

    ft = jnp.transpose(field, (1, 0, 2))

    grid = (B // _TILE,)
    out = pl.pallas_call(
        _mlp_kernel,
        grid=grid,
        in_specs=[
            pl.BlockSpec((nf, _TILE, nl), lambda i: (0, i, 0)),
            pl.BlockSpec((hid1, in_dim), lambda i: (0, 0)),
            pl.BlockSpec((hid1,), lambda i: (0,)),
            pl.BlockSpec((hid2, hid1), lambda i: (0, 0)),
            pl.BlockSpec((hid2,), lambda i: (0,)),
            pl.BlockSpec((1, hid2), lambda i: (0, 0)),
            pl.BlockSpec((1,), lambda i: (0,)),
        ],
        out_specs=pl.BlockSpec((_TILE, 1), lambda i: (i, 0)),
        out_shape=jax.ShapeDtypeStruct((B, 1), jnp.float32),
        scratch_shapes=[
            pltpu.VMEM((in_dim, hid1), jnp.bfloat16),
        ],
    )(ft, W1, b1, W2, b2, W3, b3)
    return out


# speedup vs baseline: 1.0707x; 1.0194x over previous
import jax
import jax.numpy as jnp
from jax.experimental import pallas as pl
from jax.experimental.pallas import tpu as pltpu

_TILE = 512


def _mlp_kernel(x_ref, w1_ref, b1_ref, w2_ref, b2_ref, w3_ref, b3_ref,
                o_ref, w1bf_ref):
    @pl.when(pl.program_id(0) == 0)
    def _():
        w1bf_ref[...] = w1_ref[...].astype(jnp.bfloat16)

    w2b = w2_ref[...].T.astype(jnp.bfloat16)
    w3b = w3_ref[...].T.astype(jnp.bfloat16)
    b1v = b1_ref[...].reshape(1, -1)
    b2v = b2_ref[...].reshape(1, -1)
    b3v = b3_ref[...].reshape(1, -1)

    nf = x_ref.shape[0]
    xr = jnp.concatenate(
        [x_ref[f].astype(jnp.bfloat16) for f in range(nf)], axis=1)
    h1 = jax.lax.dot_general(xr, w1bf_ref[...], (((1,), (1,)), ((), ())),
                             preferred_element_type=jnp.float32)
    h1 = jnp.maximum(h1 + b1v, 0.0).astype(jnp.bfloat16)
    h2 = jnp.dot(h1, w2b, preferred_element_type=jnp.float32)
    h2 = jnp.maximum(h2 + b2v, 0.0).astype(jnp.bfloat16)
    out = jnp.dot(h2, w3b, preferred_element_type=jnp.float32)
    o_ref[...] = out + b3v


def kernel(field, W1, b1, W2, b2, W3, b3):
    B = field.shape[0]
    nf, nl = field.shape[1], field.shape[2]
    in_dim = nf * nl
    hid1 = W1.shape[0]
    hid2 = W2.shape[0]

    ft = jnp.transpose(field, (1, 0, 2))

    grid = (B // _TILE,)
    out = pl.pallas_call(
        _mlp_kernel,
        grid=grid,
        in_specs=[
            pl.BlockSpec((nf, _TILE, nl), lambda i: (0, i, 0)),
            pl.BlockSpec((hid1, in_dim), lambda i: (0, 0)),
            pl.BlockSpec((hid1,), lambda i: (0,)),
            pl.BlockSpec((hid2, hid1), lambda i: (0, 0)),
            pl.BlockSpec((hid2,), lambda i: (0,)),
            pl.BlockSpec((1, hid2), lambda i: (0, 0)),
            pl.BlockSpec((1,), lambda i: (0,)),
        ],
        out_specs=pl.BlockSpec((_TILE, 1), lambda i: (i, 0)),
        out_shape=jax.ShapeDtypeStruct((B, 1), jnp.float32),
        scratch_shapes=[
            pltpu.VMEM((hid1, in_dim), jnp.bfloat16),
        ],
    )(ft, W1, b1, W2, b2, W3, b3)
    return out
